# baseline (speedup 1.0000x reference)
import functools

import jax
import jax.numpy as jnp
from jax.experimental import pallas as pl
from jax.experimental.pallas import tpu as pltpu

_BLOCK = 1000


def _gc_block_kernel(item_ref, user_ref, ohc_ref, Wu_ref, bu_ref,
                     Wv_ref, bv_ref, Wl_ref, bl_ref, u_out_ref, v_out_ref,
                     WuT_s, WvT_s, WlT_s, *, num_classes, hidden):
    @pl.when(pl.program_id(0) == 0)
    def _init():
        WuT_s[...] = Wu_ref[...].T
        WvT_s[...] = Wv_ref[...].T
        WlT_s[...] = Wl_ref[...].T

    x_item = item_ref[...]
    x_user = user_ref[...]
    m = ohc_ref[...]
    zu = jnp.dot(x_item, WuT_s[...], preferred_element_type=jnp.float32)
    zv = jnp.dot(x_user, WvT_s[...], preferred_element_type=jnp.float32)
    H = hidden
    un = m[:, 0:1] * (zu[:, 0:H] + bu_ref[0:1, :])
    vn = m[:, 0:1] * (zv[:, 0:H] + bv_ref[0:1, :])
    for cc in range(1, num_classes):
        un += m[:, cc:cc + 1] * (zu[:, cc * H:(cc + 1) * H] + bu_ref[cc:cc + 1, :])
        vn += m[:, cc:cc + 1] * (zv[:, cc * H:(cc + 1) * H] + bv_ref[cc:cc + 1, :])
    hu = jnp.maximum(un, 0.0)
    hv = jnp.maximum(vn, 0.0)
    ou = jnp.dot(hu, WlT_s[...], preferred_element_type=jnp.float32) + bl_ref[...]
    ov = jnp.dot(hv, WlT_s[...], preferred_element_type=jnp.float32) + bl_ref[...]
    u_out_ref[...] = jnp.maximum(ou, 0.0)
    v_out_ref[...] = jnp.maximum(ov, 0.0)


def kernel(user, item, r, c, Wu, bu, Wv, bv, Wl, bl):
    N, I = user.shape
    C, H, _ = Wu.shape
    O = Wl.shape[0]
    ohc = (r[:, None] == jnp.arange(C, dtype=r.dtype)[None, :]).astype(
        jnp.float32) * c[:, None]
    nb = N // _BLOCK
    bs_x = pl.BlockSpec((_BLOCK, I), lambda i: (i, 0))
    bs_m = pl.BlockSpec((_BLOCK, C), lambda i: (i, 0))
    bs_W = pl.BlockSpec((C * H, I), lambda i: (0, 0))
    bs_b = pl.BlockSpec((C, H), lambda i: (0, 0))
    bs_Wl = pl.BlockSpec((O, H), lambda i: (0, 0))
    bs_bl = pl.BlockSpec((1, O), lambda i: (0, 0))
    bs_out = pl.BlockSpec((_BLOCK, O), lambda i: (i, 0))
    u_out, v_out = pl.pallas_call(
        functools.partial(_gc_block_kernel, num_classes=C, hidden=H),
        grid=(nb,),
        in_specs=[bs_x, bs_x, bs_m, bs_W, bs_b, bs_W, bs_b, bs_Wl, bs_bl],
        out_specs=[bs_out, bs_out],
        out_shape=[jax.ShapeDtypeStruct((N, O), jnp.float32)] * 2,
        scratch_shapes=[
            pltpu.VMEM((I, C * H), jnp.float32),
            pltpu.VMEM((I, C * H), jnp.float32),
            pltpu.VMEM((H, O), jnp.float32),
        ],
        compiler_params=pltpu.CompilerParams(
            dimension_semantics=("arbitrary",)),
    )(item, user, ohc, Wu.reshape(C * H, I), bu, Wv.reshape(C * H, I),
      bv, Wl, bl.reshape(1, O))
    return (u_out, v_out)


# speedup vs baseline: 1.0021x; 1.0021x over previous
import functools

import jax
import jax.numpy as jnp
from jax.experimental import pallas as pl
from jax.experimental.pallas import tpu as pltpu

_BLOCK = 1000


def _gc_block_kernel(item_ref, user_ref, ohc_ref, Wu_ref, bu_ref,
                     Wv_ref, bv_ref, Wl_ref, bl_ref, u_out_ref, v_out_ref,
                     WuT_s, WvT_s, WlT_s, *, num_classes, hidden):
    @pl.when(pl.program_id(0) == 0)
    def _init():
        WuT_s[...] = Wu_ref[...].T.astype(jnp.bfloat16)
        WvT_s[...] = Wv_ref[...].T.astype(jnp.bfloat16)
        WlT_s[...] = Wl_ref[...].T.astype(jnp.bfloat16)

    x_item = item_ref[...].astype(jnp.bfloat16)
    x_user = user_ref[...].astype(jnp.bfloat16)
    m = ohc_ref[...]
    zu = jnp.dot(x_item, WuT_s[...], preferred_element_type=jnp.float32)
    zv = jnp.dot(x_user, WvT_s[...], preferred_element_type=jnp.float32)
    H = hidden
    un = m[:, 0:1] * (zu[:, 0:H] + bu_ref[0:1, :])
    vn = m[:, 0:1] * (zv[:, 0:H] + bv_ref[0:1, :])
    for cc in range(1, num_classes):
        un += m[:, cc:cc + 1] * (zu[:, cc * H:(cc + 1) * H] + bu_ref[cc:cc + 1, :])
        vn += m[:, cc:cc + 1] * (zv[:, cc * H:(cc + 1) * H] + bv_ref[cc:cc + 1, :])
    hu = jnp.maximum(un, 0.0).astype(jnp.bfloat16)
    hv = jnp.maximum(vn, 0.0).astype(jnp.bfloat16)
    ou = jnp.dot(hu, WlT_s[...], preferred_element_type=jnp.float32) + bl_ref[...]
    ov = jnp.dot(hv, WlT_s[...], preferred_element_type=jnp.float32) + bl_ref[...]
    u_out_ref[...] = jnp.maximum(ou, 0.0)
    v_out_ref[...] = jnp.maximum(ov, 0.0)


def kernel(user, item, r, c, Wu, bu, Wv, bv, Wl, bl):
    N, I = user.shape
    C, H, _ = Wu.shape
    O = Wl.shape[0]
    ohc = (r[:, None] == jnp.arange(C, dtype=r.dtype)[None, :]).astype(
        jnp.float32) * c[:, None]
    nb = N // _BLOCK
    bs_x = pl.BlockSpec((_BLOCK, I), lambda i: (i, 0))
    bs_m = pl.BlockSpec((_BLOCK, C), lambda i: (i, 0))
    bs_W = pl.BlockSpec((C * H, I), lambda i: (0, 0))
    bs_b = pl.BlockSpec((C, H), lambda i: (0, 0))
    bs_Wl = pl.BlockSpec((O, H), lambda i: (0, 0))
    bs_bl = pl.BlockSpec((1, O), lambda i: (0, 0))
    bs_out = pl.BlockSpec((_BLOCK, O), lambda i: (i, 0))
    u_out, v_out = pl.pallas_call(
        functools.partial(_gc_block_kernel, num_classes=C, hidden=H),
        grid=(nb,),
        in_specs=[bs_x, bs_x, bs_m, bs_W, bs_b, bs_W, bs_b, bs_Wl, bs_bl],
        out_specs=[bs_out, bs_out],
        out_shape=[jax.ShapeDtypeStruct((N, O), jnp.float32)] * 2,
        scratch_shapes=[
            pltpu.VMEM((I, C * H), jnp.bfloat16),
            pltpu.VMEM((I, C * H), jnp.bfloat16),
            pltpu.VMEM((H, O), jnp.bfloat16),
        ],
        compiler_params=pltpu.CompilerParams(
            dimension_semantics=("arbitrary",)),
    )(item, user, ohc, Wu.reshape(C * H, I), bu, Wv.reshape(C * H, I),
      bv, Wl, bl.reshape(1, O))
    return (u_out, v_out)
